# baseline (speedup 1.0000x reference)
import jax
import jax.numpy as jnp
from jax import lax
from jax.experimental import pallas as pl
from jax.experimental.pallas import tpu as pltpu
from jax.experimental.pallas import tpu_sc as plsc

N = 10000
E = 320000
D = 128
NCLS = 6
NC = 2
NS = 16
EPT = E // NS
CH = 128
G = 16
NGRP = -(-EPT // (G * CH))
NGRPP = NGRP + 1
EPAD = NGRPP * G * CH
ZR = 640
NPAD = NS * ZR
DUMMY = N + 8

_MESH = plsc.VectorSubcoreMesh(core_axis_name="c", subcore_axis_name="s")


def _deg_body(sidx, didx, ones, zeros, out, d_src, d_dst, sidx_v, didx_v,
              ones_v):
    c = lax.axis_index("c")
    s = lax.axis_index("s")
    pltpu.sync_copy(ones, ones_v)
    pltpu.sync_copy(zeros, d_src.at[pl.ds(s * ZR, ZR)])
    pltpu.sync_copy(zeros, d_dst.at[pl.ds(s * ZR, ZR)])
    plsc.subcore_barrier()

    def grp(g, carry):
        pltpu.sync_copy(sidx.at[c, s, g], sidx_v)
        pltpu.sync_copy(didx.at[c, s, g], didx_v)

        def body(j, carry2):
            pltpu.sync_copy(ones_v, d_src.at[sidx_v.at[j]], add=True)
            pltpu.sync_copy(ones_v, d_dst.at[didx_v.at[j]], add=True)
            return carry2

        lax.fori_loop(0, G, body, 0)
        return carry

    lax.fori_loop(0, NGRP, grp, 0)
    plsc.subcore_barrier()
    pltpu.sync_copy(d_src.at[pl.ds(s * ZR, ZR)],
                    out.at[pl.ds((c * 2 + 0) * NPAD + s * ZR, ZR)])
    pltpu.sync_copy(d_dst.at[pl.ds(s * ZR, ZR)],
                    out.at[pl.ds((c * 2 + 1) * NPAD + s * ZR, ZR)])


_deg_call = pl.kernel(
    _deg_body,
    out_type=jax.ShapeDtypeStruct((NC * 2 * NPAD,), jnp.float32),
    mesh=_MESH,
    scratch_types=[
        pltpu.VMEM_SHARED((NPAD,), jnp.float32),
        pltpu.VMEM_SHARED((NPAD,), jnp.float32),
        pltpu.VMEM((G, CH), jnp.int32),
        pltpu.VMEM((G, CH), jnp.int32),
        pltpu.VMEM((CH,), jnp.float32),
    ],
)


def _agg_body(y, sidx, didx, zeros, out, agg_sh, sidx_v, didx_v, rows_v,
              gsem00, gsem01, gsem10, gsem11, ssem0, ssem1, isem):
    c = lax.axis_index("c")
    s = lax.axis_index("s")
    gs = ((gsem00, gsem01), (gsem10, gsem11))
    ss = (ssem0, ssem1)
    pltpu.sync_copy(zeros, agg_sh.at[pl.ds(s * ZR, ZR)])
    pltpu.async_copy(sidx.at[c, s, 0], sidx_v.at[0], isem)
    pltpu.async_copy(didx.at[c, s, 0], didx_v.at[0], isem)
    plsc.subcore_barrier()

    H = CH // 2

    def fire_halves(gb, j, b):
        return tuple(
            pltpu.async_copy(y.at[sidx_v.at[gb, j, pl.ds(h * H, H)]],
                             rows_v.at[b, pl.ds(h * H, H)], gs[b][h])
            for h in (0, 1))

    def ring(gb, g):
        cps = {}
        scs = {}
        cps[0] = fire_halves(gb, 0, 0)
        cps[1] = fire_halves(gb, 1, 1)
        for j in range(G):
            b = j % 2
            cps[j][0].wait()
            cps[j][1].wait()
            scs[j] = pltpu.async_copy(rows_v.at[b],
                                      agg_sh.at[didx_v.at[gb, j]],
                                      ss[b], add=True)
            if j + 2 < G:
                scs[j].wait()
                cps[j + 2] = fire_halves(gb, j + 2, b)
        scs[G - 2].wait()
        scs[G - 1].wait()

    def super_grp(t, carry):
        g0 = 2 * t
        pltpu.make_async_copy(sidx.at[c, s, 0], sidx_v.at[0], isem).wait()
        pltpu.make_async_copy(didx.at[c, s, 0], didx_v.at[0], isem).wait()
        pltpu.async_copy(sidx.at[c, s, g0 + 1], sidx_v.at[1], isem)
        pltpu.async_copy(didx.at[c, s, g0 + 1], didx_v.at[1], isem)
        ring(0, g0)
        pltpu.make_async_copy(sidx.at[c, s, 0], sidx_v.at[1], isem).wait()
        pltpu.make_async_copy(didx.at[c, s, 0], didx_v.at[1], isem).wait()
        pltpu.async_copy(sidx.at[c, s, g0 + 2], sidx_v.at[0], isem)
        pltpu.async_copy(didx.at[c, s, g0 + 2], didx_v.at[0], isem)
        ring(1, g0 + 1)
        return carry

    lax.fori_loop(0, NGRP // 2, super_grp, 0)
    pltpu.make_async_copy(sidx.at[c, s, 0], sidx_v.at[0], isem).wait()
    pltpu.make_async_copy(didx.at[c, s, 0], didx_v.at[0], isem).wait()
    plsc.subcore_barrier()
    pltpu.sync_copy(agg_sh.at[pl.ds(s * ZR, ZR)], out.at[c, s])


_agg_call = pl.kernel(
    _agg_body,
    out_type=jax.ShapeDtypeStruct((NC, NS, ZR, D), jnp.float32),
    mesh=_MESH,
    scratch_types=[
        pltpu.VMEM_SHARED((NPAD, D), jnp.float32),
        pltpu.VMEM((2, G, CH), jnp.int32),
        pltpu.VMEM((2, G, CH), jnp.int32),
        pltpu.VMEM((2, CH, D), jnp.float32),
        pltpu.SemaphoreType.DMA,
        pltpu.SemaphoreType.DMA,
        pltpu.SemaphoreType.DMA,
        pltpu.SemaphoreType.DMA,
        pltpu.SemaphoreType.DMA,
        pltpu.SemaphoreType.DMA,
        pltpu.SemaphoreType.DMA,
    ],
)


_R = 2048


def _mm1_body(x_ref, ds_ref, w_ref, o_ref):
    ns = lax.rsqrt(jnp.clip(ds_ref[...], 1.0, None))
    o_ref[...] = jnp.dot(x_ref[...] * ns, w_ref[...],
                         preferred_element_type=jnp.float32)


def _mm1(x, ds, w):
    return pl.pallas_call(
        _mm1_body,
        grid=(2 * NPAD // _R,),
        in_specs=[
            pl.BlockSpec((_R, D), lambda i: (i, 0)),
            pl.BlockSpec((_R, 1), lambda i: (i, 0)),
            pl.BlockSpec((D, D), lambda i: (0, 0)),
        ],
        out_specs=pl.BlockSpec((_R, D), lambda i: (i, 0)),
        out_shape=jax.ShapeDtypeStruct((2 * NPAD, D), jnp.float32),
    )(x, ds, w)


def _mm2_body(a_ref, ds_ref, di_ref, w_ref, b_ref, o_ref):
    ns = lax.rsqrt(jnp.clip(ds_ref[...], 1.0, None))
    nd = lax.rsqrt(jnp.clip(di_ref[...], 1.0, None))
    h = jnp.maximum(a_ref[...] * nd + b_ref[...], 0.0) * ns
    o_ref[...] = jnp.dot(h, w_ref[...], preferred_element_type=jnp.float32)


def _mm2(a, ds, di, w, b):
    return pl.pallas_call(
        _mm2_body,
        grid=(2 * NPAD // _R,),
        in_specs=[
            pl.BlockSpec((_R, D), lambda i: (i, 0)),
            pl.BlockSpec((_R, 1), lambda i: (i, 0)),
            pl.BlockSpec((_R, 1), lambda i: (i, 0)),
            pl.BlockSpec((D, D), lambda i: (0, 0)),
            pl.BlockSpec((1, D), lambda i: (0, 0)),
        ],
        out_specs=pl.BlockSpec((_R, D), lambda i: (i, 0)),
        out_shape=jax.ShapeDtypeStruct((2 * NPAD, D), jnp.float32),
    )(a, ds, di, w, b)


_NB = NPAD // _R


def _pool_body(a_ref, di_ref, b_ref, o_ref):
    j = pl.program_id(1)
    nd = lax.rsqrt(jnp.clip(di_ref[...], 1.0, None))
    h = jnp.maximum(a_ref[...] * nd + b_ref[...], 0.0)
    row = j * _R + lax.broadcasted_iota(jnp.int32, (_R, 1), 0)
    h = jnp.where(row < N, h, 0.0)
    part = (jnp.sum(h, axis=0, keepdims=True) * (1.0 / N)).reshape(1, 1, D)

    @pl.when(j == 0)
    def _():
        o_ref[...] = part

    @pl.when(j > 0)
    def _():
        o_ref[...] += part


def _pool(a, di, b):
    return pl.pallas_call(
        _pool_body,
        grid=(2, _NB),
        in_specs=[
            pl.BlockSpec((_R, D), lambda g, j: (g * _NB + j, 0)),
            pl.BlockSpec((_R, 1), lambda g, j: (g * _NB + j, 0)),
            pl.BlockSpec((1, D), lambda g, j: (0, 0)),
        ],
        out_specs=pl.BlockSpec((1, 1, D), lambda g, j: (g, 0, 0)),
        out_shape=jax.ShapeDtypeStruct((2, 1, D), jnp.float32),
    )(a, di, b).reshape(2, D)


def _head_body(m_ref, w1_ref, w2_ref, bc_ref, o_ref):
    logits = (jnp.dot(m_ref[0:1, :], w1_ref[...],
                      preferred_element_type=jnp.float32)
              + jnp.dot(m_ref[1:2, :], w2_ref[...],
                        preferred_element_type=jnp.float32)
              + bc_ref[...])
    z = logits - jnp.max(logits, axis=-1, keepdims=True)
    o_ref[...] = z - jnp.log(jnp.sum(jnp.exp(z), axis=-1, keepdims=True))


def _head(m, w1, w2, bc):
    return pl.pallas_call(
        _head_body,
        in_specs=[
            pl.BlockSpec((2, D), lambda: (0, 0)),
            pl.BlockSpec((D, NCLS), lambda: (0, 0)),
            pl.BlockSpec((D, NCLS), lambda: (0, 0)),
            pl.BlockSpec((1, NCLS), lambda: (0, 0)),
        ],
        out_specs=pl.BlockSpec((1, NCLS), lambda: (0, 0)),
        out_shape=jax.ShapeDtypeStruct((1, NCLS), jnp.float32),
    )(m, w1, w2, bc)


def _pad_idx(a, padval):
    a = a.reshape(2, NS, EPT)
    a = jnp.pad(a, ((0, 0), (0, 0), (0, EPAD - EPT)), constant_values=padval)
    return a.reshape(2, NS, NGRPP, G, CH)


def kernel(x1, edge_index1, x2, edge_index2, W1, b1, W2, b2, W3, b3, Wc, bc):
    src = jnp.stack([edge_index1[0], edge_index2[0]]).astype(jnp.int32)
    dst = jnp.stack([edge_index1[1], edge_index2[1]]).astype(jnp.int32)
    src_loc = _pad_idx(src, DUMMY)
    dst_loc = _pad_idx(dst, DUMMY)
    goff = (jnp.arange(2, dtype=jnp.int32) * NPAD).reshape(2, 1, 1, 1, 1)
    src_glb = _pad_idx(src, 0) + goff
    ones1 = jnp.ones((CH,), jnp.float32)
    zeros1 = jnp.zeros((ZR,), jnp.float32)
    zerosD = jnp.zeros((ZR, D), jnp.float32)

    deg = _deg_call(src_loc, dst_loc, ones1, zeros1)
    deg = deg.reshape(2, 2, NPAD)
    ds = deg[:, 0].reshape(2 * NPAD, 1)
    di = deg[:, 1].reshape(2 * NPAD, 1)

    def agg(y):
        return _agg_call(y, src_glb, dst_loc, zerosD).reshape(2 * NPAD, D)

    x = jnp.zeros((2, NPAD, D), jnp.float32)
    x = x.at[:, :N].set(jnp.stack([x1, x2])).reshape(2 * NPAD, D)
    a = agg(_mm1(x, ds, W1))
    a = agg(_mm2(a, ds, di, W2, b1.reshape(1, D)))
    a = agg(_mm2(a, ds, di, W3, b2.reshape(1, D)))
    m = _pool(a, di, b3.reshape(1, D))
    return _head(m, Wc[:D], Wc[D:], bc.reshape(1, NCLS))


# speedup vs baseline: 1.0057x; 1.0057x over previous
import jax
import jax.numpy as jnp
from jax import lax
from jax.experimental import pallas as pl
from jax.experimental.pallas import tpu as pltpu
from jax.experimental.pallas import tpu_sc as plsc

N = 10000
E = 320000
D = 128
NCLS = 6
NC = 2
NS = 16
EPT = E // NS
CH = 128
G = 20
NGRP = -(-EPT // (G * CH))
NGRPP = NGRP + 1
EPAD = NGRPP * G * CH
ZR = 640
NPAD = NS * ZR
DUMMY = N + 8

_MESH = plsc.VectorSubcoreMesh(core_axis_name="c", subcore_axis_name="s")


def _deg_body(sidx, didx, ones, zeros, out, d_src, d_dst, sidx_v, didx_v,
              ones_v):
    c = lax.axis_index("c")
    s = lax.axis_index("s")
    pltpu.sync_copy(ones, ones_v)
    pltpu.sync_copy(zeros, d_src.at[pl.ds(s * ZR, ZR)])
    pltpu.sync_copy(zeros, d_dst.at[pl.ds(s * ZR, ZR)])
    plsc.subcore_barrier()

    def grp(g, carry):
        pltpu.sync_copy(sidx.at[c, s, g], sidx_v)
        pltpu.sync_copy(didx.at[c, s, g], didx_v)

        def body(j, carry2):
            pltpu.sync_copy(ones_v, d_src.at[sidx_v.at[j]], add=True)
            pltpu.sync_copy(ones_v, d_dst.at[didx_v.at[j]], add=True)
            return carry2

        lax.fori_loop(0, G, body, 0)
        return carry

    lax.fori_loop(0, NGRP, grp, 0)
    plsc.subcore_barrier()
    pltpu.sync_copy(d_src.at[pl.ds(s * ZR, ZR)],
                    out.at[pl.ds((c * 2 + 0) * NPAD + s * ZR, ZR)])
    pltpu.sync_copy(d_dst.at[pl.ds(s * ZR, ZR)],
                    out.at[pl.ds((c * 2 + 1) * NPAD + s * ZR, ZR)])


_deg_call = pl.kernel(
    _deg_body,
    out_type=jax.ShapeDtypeStruct((NC * 2 * NPAD,), jnp.float32),
    mesh=_MESH,
    scratch_types=[
        pltpu.VMEM_SHARED((NPAD,), jnp.float32),
        pltpu.VMEM_SHARED((NPAD,), jnp.float32),
        pltpu.VMEM((G, CH), jnp.int32),
        pltpu.VMEM((G, CH), jnp.int32),
        pltpu.VMEM((CH,), jnp.float32),
    ],
)


def _agg_body(y, sidx, didx, zeros, out, agg_sh, sidx_v, didx_v, rows_v,
              gsem00, gsem01, gsem10, gsem11, ssem0, ssem1, isem):
    c = lax.axis_index("c")
    s = lax.axis_index("s")
    gs = ((gsem00, gsem01), (gsem10, gsem11))
    ss = (ssem0, ssem1)
    pltpu.sync_copy(zeros, agg_sh.at[pl.ds(s * ZR, ZR)])
    pltpu.async_copy(sidx.at[c, s, 0], sidx_v.at[0], isem)
    pltpu.async_copy(didx.at[c, s, 0], didx_v.at[0], isem)
    plsc.subcore_barrier()

    H = CH // 2

    def fire_halves(gb, j, b):
        return tuple(
            pltpu.async_copy(y.at[sidx_v.at[gb, j, pl.ds(h * H, H)]],
                             rows_v.at[b, pl.ds(h * H, H)], gs[b][h])
            for h in (0, 1))

    def ring(gb, g):
        cps = {}
        scs = {}
        cps[0] = fire_halves(gb, 0, 0)
        cps[1] = fire_halves(gb, 1, 1)
        for j in range(G):
            b = j % 2
            cps[j][0].wait()
            cps[j][1].wait()
            scs[j] = pltpu.async_copy(rows_v.at[b],
                                      agg_sh.at[didx_v.at[gb, j]],
                                      ss[b], add=True)
            if j + 2 < G:
                scs[j].wait()
                cps[j + 2] = fire_halves(gb, j + 2, b)
        scs[G - 2].wait()
        scs[G - 1].wait()

    def super_grp(t, carry):
        g0 = 2 * t
        pltpu.make_async_copy(sidx.at[c, s, 0], sidx_v.at[0], isem).wait()
        pltpu.make_async_copy(didx.at[c, s, 0], didx_v.at[0], isem).wait()
        pltpu.async_copy(sidx.at[c, s, g0 + 1], sidx_v.at[1], isem)
        pltpu.async_copy(didx.at[c, s, g0 + 1], didx_v.at[1], isem)
        ring(0, g0)
        pltpu.make_async_copy(sidx.at[c, s, 0], sidx_v.at[1], isem).wait()
        pltpu.make_async_copy(didx.at[c, s, 0], didx_v.at[1], isem).wait()
        pltpu.async_copy(sidx.at[c, s, g0 + 2], sidx_v.at[0], isem)
        pltpu.async_copy(didx.at[c, s, g0 + 2], didx_v.at[0], isem)
        ring(1, g0 + 1)
        return carry

    lax.fori_loop(0, NGRP // 2, super_grp, 0)
    pltpu.make_async_copy(sidx.at[c, s, 0], sidx_v.at[0], isem).wait()
    pltpu.make_async_copy(didx.at[c, s, 0], didx_v.at[0], isem).wait()
    plsc.subcore_barrier()
    pltpu.sync_copy(agg_sh.at[pl.ds(s * ZR, ZR)], out.at[c, s])


_agg_call = pl.kernel(
    _agg_body,
    out_type=jax.ShapeDtypeStruct((NC, NS, ZR, D), jnp.float32),
    mesh=_MESH,
    scratch_types=[
        pltpu.VMEM_SHARED((NPAD, D), jnp.float32),
        pltpu.VMEM((2, G, CH), jnp.int32),
        pltpu.VMEM((2, G, CH), jnp.int32),
        pltpu.VMEM((2, CH, D), jnp.float32),
        pltpu.SemaphoreType.DMA,
        pltpu.SemaphoreType.DMA,
        pltpu.SemaphoreType.DMA,
        pltpu.SemaphoreType.DMA,
        pltpu.SemaphoreType.DMA,
        pltpu.SemaphoreType.DMA,
        pltpu.SemaphoreType.DMA,
    ],
)


_R = 2048


def _mm1_body(x_ref, ds_ref, w_ref, o_ref):
    ns = lax.rsqrt(jnp.clip(ds_ref[...], 1.0, None))
    o_ref[...] = jnp.dot(x_ref[...] * ns, w_ref[...],
                         preferred_element_type=jnp.float32)


def _mm1(x, ds, w):
    return pl.pallas_call(
        _mm1_body,
        grid=(2 * NPAD // _R,),
        in_specs=[
            pl.BlockSpec((_R, D), lambda i: (i, 0)),
            pl.BlockSpec((_R, 1), lambda i: (i, 0)),
            pl.BlockSpec((D, D), lambda i: (0, 0)),
        ],
        out_specs=pl.BlockSpec((_R, D), lambda i: (i, 0)),
        out_shape=jax.ShapeDtypeStruct((2 * NPAD, D), jnp.float32),
    )(x, ds, w)


def _mm2_body(a_ref, ds_ref, di_ref, w_ref, b_ref, o_ref):
    ns = lax.rsqrt(jnp.clip(ds_ref[...], 1.0, None))
    nd = lax.rsqrt(jnp.clip(di_ref[...], 1.0, None))
    h = jnp.maximum(a_ref[...] * nd + b_ref[...], 0.0) * ns
    o_ref[...] = jnp.dot(h, w_ref[...], preferred_element_type=jnp.float32)


def _mm2(a, ds, di, w, b):
    return pl.pallas_call(
        _mm2_body,
        grid=(2 * NPAD // _R,),
        in_specs=[
            pl.BlockSpec((_R, D), lambda i: (i, 0)),
            pl.BlockSpec((_R, 1), lambda i: (i, 0)),
            pl.BlockSpec((_R, 1), lambda i: (i, 0)),
            pl.BlockSpec((D, D), lambda i: (0, 0)),
            pl.BlockSpec((1, D), lambda i: (0, 0)),
        ],
        out_specs=pl.BlockSpec((_R, D), lambda i: (i, 0)),
        out_shape=jax.ShapeDtypeStruct((2 * NPAD, D), jnp.float32),
    )(a, ds, di, w, b)


_NB = NPAD // _R


def _pool_body(a_ref, di_ref, b_ref, o_ref):
    j = pl.program_id(1)
    nd = lax.rsqrt(jnp.clip(di_ref[...], 1.0, None))
    h = jnp.maximum(a_ref[...] * nd + b_ref[...], 0.0)
    row = j * _R + lax.broadcasted_iota(jnp.int32, (_R, 1), 0)
    h = jnp.where(row < N, h, 0.0)
    part = (jnp.sum(h, axis=0, keepdims=True) * (1.0 / N)).reshape(1, 1, D)

    @pl.when(j == 0)
    def _():
        o_ref[...] = part

    @pl.when(j > 0)
    def _():
        o_ref[...] += part


def _pool(a, di, b):
    return pl.pallas_call(
        _pool_body,
        grid=(2, _NB),
        in_specs=[
            pl.BlockSpec((_R, D), lambda g, j: (g * _NB + j, 0)),
            pl.BlockSpec((_R, 1), lambda g, j: (g * _NB + j, 0)),
            pl.BlockSpec((1, D), lambda g, j: (0, 0)),
        ],
        out_specs=pl.BlockSpec((1, 1, D), lambda g, j: (g, 0, 0)),
        out_shape=jax.ShapeDtypeStruct((2, 1, D), jnp.float32),
    )(a, di, b).reshape(2, D)


def _head_body(m_ref, w1_ref, w2_ref, bc_ref, o_ref):
    logits = (jnp.dot(m_ref[0:1, :], w1_ref[...],
                      preferred_element_type=jnp.float32)
              + jnp.dot(m_ref[1:2, :], w2_ref[...],
                        preferred_element_type=jnp.float32)
              + bc_ref[...])
    z = logits - jnp.max(logits, axis=-1, keepdims=True)
    o_ref[...] = z - jnp.log(jnp.sum(jnp.exp(z), axis=-1, keepdims=True))


def _head(m, w1, w2, bc):
    return pl.pallas_call(
        _head_body,
        in_specs=[
            pl.BlockSpec((2, D), lambda: (0, 0)),
            pl.BlockSpec((D, NCLS), lambda: (0, 0)),
            pl.BlockSpec((D, NCLS), lambda: (0, 0)),
            pl.BlockSpec((1, NCLS), lambda: (0, 0)),
        ],
        out_specs=pl.BlockSpec((1, NCLS), lambda: (0, 0)),
        out_shape=jax.ShapeDtypeStruct((1, NCLS), jnp.float32),
    )(m, w1, w2, bc)


def _pad_idx(a, padval):
    a = a.reshape(2, NS, EPT)
    a = jnp.pad(a, ((0, 0), (0, 0), (0, EPAD - EPT)), constant_values=padval)
    return a.reshape(2, NS, NGRPP, G, CH)


def kernel(x1, edge_index1, x2, edge_index2, W1, b1, W2, b2, W3, b3, Wc, bc):
    src = jnp.stack([edge_index1[0], edge_index2[0]]).astype(jnp.int32)
    dst = jnp.stack([edge_index1[1], edge_index2[1]]).astype(jnp.int32)
    src_loc = _pad_idx(src, DUMMY)
    dst_loc = _pad_idx(dst, DUMMY)
    goff = (jnp.arange(2, dtype=jnp.int32) * NPAD).reshape(2, 1, 1, 1, 1)
    src_glb = _pad_idx(src, 0) + goff
    ones1 = jnp.ones((CH,), jnp.float32)
    zeros1 = jnp.zeros((ZR,), jnp.float32)
    zerosD = jnp.zeros((ZR, D), jnp.float32)

    deg = _deg_call(src_loc, dst_loc, ones1, zeros1)
    deg = deg.reshape(2, 2, NPAD)
    ds = deg[:, 0].reshape(2 * NPAD, 1)
    di = deg[:, 1].reshape(2 * NPAD, 1)

    def agg(y):
        return _agg_call(y, src_glb, dst_loc, zerosD).reshape(2 * NPAD, D)

    x = jnp.zeros((2, NPAD, D), jnp.float32)
    x = x.at[:, :N].set(jnp.stack([x1, x2])).reshape(2 * NPAD, D)
    a = agg(_mm1(x, ds, W1))
    a = agg(_mm2(a, ds, di, W2, b1.reshape(1, D)))
    a = agg(_mm2(a, ds, di, W3, b2.reshape(1, D)))
    m = _pool(a, di, b3.reshape(1, D))
    return _head(m, Wc[:D], Wc[D:], bc.reshape(1, NCLS))
